# Initial kernel scaffold; baseline (speedup 1.0000x reference)
import jax
import jax.numpy as jnp
from jax.experimental import pallas as pl


def kernel(x, edge_index, edge_weight, Wcz, bcz, Wcr, bcr, Wch, bch, Wlz, blz, Wlr, blr, Wlh, blh, att, Wlin, blin):
    raise NotImplementedError("write your pallas kernel here")


# speedup vs baseline: 213.6771x; 213.6771x over previous
import functools

import jax
import jax.numpy as jnp
from jax import lax
from jax.experimental import pallas as pl
from jax.experimental.pallas import tpu as pltpu
from jax.experimental.pallas import tpu_sc as plsc

N = 50000
E = 800000
PERIODS = 12
HID = 32

NP = 50176
NPT = NP // 16
EP = 819200
ROWS = EP // 128
NW = 32
ROWS_W = ROWS // NW
KB = 40
NG = ROWS_W // KB

_mesh = plsc.VectorSubcoreMesh(core_axis_name="c", subcore_axis_name="s")


@functools.partial(
    pl.kernel,
    out_type=jax.ShapeDtypeStruct((2 * NP,), jnp.float32),
    mesh=_mesh,
    scratch_types=[
        pltpu.VMEM_SHARED((NP,), jnp.float32),
        pltpu.VMEM((KB, 128), jnp.int32),
        pltpu.VMEM((KB, 128), jnp.float32),
        pltpu.VMEM((NPT,), jnp.float32),
    ],
    compiler_params=pltpu.CompilerParams(needs_layout_passes=False),
)
def _deg_kernel(dst_hbm, ew_hbm, out_hbm, deg_sh, dstbuf, ewbuf, zbuf):
    c = lax.axis_index("c")
    s = lax.axis_index("s")
    wid = s * 2 + c
    zero16 = jnp.zeros((16,), jnp.float32)

    def zb(i, _):
        zbuf[pl.ds(i * 16, 16)] = zero16
        return 0

    lax.fori_loop(0, NPT // 16, zb, 0)
    pltpu.sync_copy(zbuf, deg_sh.at[pl.ds(s * NPT, NPT)])
    plsc.subcore_barrier()

    def grp(g, _):
        base = wid * ROWS_W + g * KB
        pltpu.sync_copy(dst_hbm.at[pl.ds(base, KB)], dstbuf)
        pltpu.sync_copy(ew_hbm.at[pl.ds(base, KB)], ewbuf)

        def row(r, _):
            pltpu.sync_copy(ewbuf.at[r], deg_sh.at[dstbuf.at[r]], add=True)
            return 0

        lax.fori_loop(0, KB, row, 0)
        return 0

    lax.fori_loop(0, NG, grp, 0)
    plsc.subcore_barrier()
    off = pl.multiple_of(c * NP + s * NPT, 8)
    pltpu.sync_copy(deg_sh.at[pl.ds(s * NPT, NPT)], zbuf)
    pltpu.sync_copy(zbuf, out_hbm.at[pl.ds(off, NPT)])


def _dinv_body(deg_ref, out_ref):
    out_ref[...] = lax.rsqrt(deg_ref[0] + deg_ref[1] + 1.0)


_dinv_call = pl.pallas_call(
    _dinv_body,
    out_shape=jax.ShapeDtypeStruct((NP // 128, 128), jnp.float32),
)


@functools.partial(
    pl.kernel,
    out_type=jax.ShapeDtypeStruct((2, NP, 16), jnp.float32),
    mesh=_mesh,
    scratch_types=[
        pltpu.VMEM_SHARED((NP, 16), jnp.float32),
        pltpu.VMEM_SHARED((NP,), jnp.float32),
        pltpu.VMEM((KB, 128), jnp.int32),
        pltpu.VMEM((KB, 128), jnp.int32),
        pltpu.VMEM((KB, 128), jnp.float32),
        pltpu.VMEM((128,), jnp.float32),
        pltpu.VMEM((128, 16), jnp.float32),
        pltpu.VMEM((112, 16), jnp.float32),
        pltpu.VMEM((NPT // 2,), jnp.float32),
    ],
    compiler_params=pltpu.CompilerParams(
        needs_layout_passes=False, use_tc_tiling_on_sc=False),
)
def _edge_kernel(src_hbm, dst_hbm, ew_hbm, dinv_hbm, x_hbm, out_hbm,
                 s_sh, dinv_sh, srcbuf, dstbuf, ewbuf, drow, xrows, zbuf, dbuf):
    c = lax.axis_index("c")
    s = lax.axis_index("s")
    wid = s * 2 + c
    zero16 = jnp.zeros((16,), jnp.float32)

    def zb(i, _):
        zbuf[i, :] = zero16
        return 0

    lax.fori_loop(0, 112, zb, 0)
    for q in range(28):
        pltpu.sync_copy(zbuf, s_sh.at[pl.ds(s * NPT + q * 112, 112), :])
    for q in range(2):
        pltpu.sync_copy(dinv_hbm.at[pl.ds(s * NPT + q * (NPT // 2), NPT // 2)], dbuf)
        pltpu.sync_copy(dbuf, dinv_sh.at[pl.ds(s * NPT + q * (NPT // 2), NPT // 2)])
    plsc.subcore_barrier()

    def grp(g, _):
        base = wid * ROWS_W + g * KB
        pltpu.sync_copy(src_hbm.at[pl.ds(base, KB)], srcbuf)
        pltpu.sync_copy(dst_hbm.at[pl.ds(base, KB)], dstbuf)
        pltpu.sync_copy(ew_hbm.at[pl.ds(base, KB)], ewbuf)

        def row(r, _):
            pltpu.sync_copy(dinv_sh.at[srcbuf.at[r]], drow)

            def wc(l, _):
                off = l * 16
                ewbuf[r, pl.ds(off, 16)] = (ewbuf[r, pl.ds(off, 16)]
                                            * drow[pl.ds(off, 16)])
                return 0

            lax.fori_loop(0, 8, wc, 0)
            pltpu.sync_copy(x_hbm.at[srcbuf.at[r]], xrows)

            def sc(j, _):
                jb = j * 16
                w16 = ewbuf[r, pl.ds(jb, 16)]
                for u in range(16):
                    xrows[jb + u, :] = xrows[jb + u, :] * w16[u]
                return 0

            lax.fori_loop(0, 8, sc, 0)
            pltpu.sync_copy(xrows, s_sh.at[dstbuf.at[r]], add=True)
            return 0

        lax.fori_loop(0, KB, row, 0)
        return 0

    lax.fori_loop(0, NG, grp, 0)
    plsc.subcore_barrier()
    for q in range(28):
        pltpu.sync_copy(s_sh.at[pl.ds(s * NPT + q * 112, 112), :], zbuf)
        pltpu.sync_copy(zbuf, out_hbm.at[c, pl.ds(s * NPT + q * 112, 112), :])


BB = 1792
NBLK = NP // BB


def _dense_body(p_ref, st_ref, xt_ref, dinv_ref, out_ref):
    dinv = dinv_ref[...]
    t = st_ref[0] + st_ref[1]
    sfull = dinv * t + (dinv * dinv) * xt_ref[...]
    az = p_ref[0, :].reshape(HID, 1)
    cz = p_ref[1, :].reshape(HID, 1)
    ah = p_ref[2, :].reshape(HID, 1)
    ch = p_ref[3, :].reshape(HID, 1)
    wl = p_ref[4, :].reshape(HID, 1)
    acc = jnp.zeros((HID, BB), jnp.float32)
    for p in range(PERIODS):
        sp = sfull[p:p + 1, :]
        z = jax.nn.sigmoid(az * sp + cz)
        ht = jnp.tanh(ah * sp + ch)
        acc = acc + p_ref[5, p] * ((1.0 - z) * ht)
    h = jnp.maximum(acc, 0.0)
    out_ref[...] = jnp.sum(h * wl, axis=0, keepdims=True) + p_ref[6, 0]


_dense_call = pl.pallas_call(
    _dense_body,
    grid=(NBLK,),
    in_specs=[
        pl.BlockSpec((8, HID), lambda i: (0, 0)),
        pl.BlockSpec((2, 16, BB), lambda i: (0, 0, i)),
        pl.BlockSpec((16, BB), lambda i: (0, i)),
        pl.BlockSpec((1, BB), lambda i: (0, i)),
    ],
    out_specs=pl.BlockSpec((1, BB), lambda i: (0, i)),
    out_shape=jax.ShapeDtypeStruct((1, NP), jnp.float32),
)


def kernel(x, edge_index, edge_weight, Wcz, bcz, Wcr, bcr, Wch, bch,
           Wlz, blz, Wlr, blr, Wlh, blh, att, Wlin, blin):
    src = edge_index[0]
    dst = edge_index[1]
    pad = EP - E
    src_p = jnp.concatenate([src, jnp.zeros((pad,), jnp.int32)])
    dst_p = jnp.concatenate([dst, (jnp.arange(pad, dtype=jnp.int32) * 41) % N])
    ew_p = jnp.concatenate([edge_weight, jnp.zeros((pad,), jnp.float32)])
    src2 = src_p.reshape(ROWS, 128)
    dst2 = dst_p.reshape(ROWS, 128)
    ew2 = ew_p.reshape(ROWS, 128)
    x_pad = jnp.pad(x, ((0, NP - N), (0, 16 - PERIODS)))

    deg2 = _deg_kernel(dst2, ew2)
    dinv = _dinv_call(deg2.reshape(2, NP // 128, 128)).reshape(NP)
    s2 = _edge_kernel(src2, dst2, ew2, dinv, x_pad)

    wlz_t = Wlz[:HID]
    wlh_t = Wlh[:HID]
    az = (Wcz @ wlz_t)[0]
    cz = bcz @ wlz_t + blz
    ah = (Wch @ wlh_t)[0]
    ch = bch @ wlh_t + blh
    probs = jax.nn.softmax(att)
    params = jnp.stack([
        az, cz, ah, ch, Wlin[:, 0],
        jnp.pad(probs, (0, HID - PERIODS)),
        jnp.full((HID,), blin[0], jnp.float32),
        jnp.zeros((HID,), jnp.float32),
    ]).astype(jnp.float32)

    st2 = jnp.transpose(s2, (0, 2, 1))
    out_t = _dense_call(params, st2, x_pad.T, dinv.reshape(1, NP))
    return out_t[0, :N].reshape(N, 1)


# speedup vs baseline: 293.0554x; 1.3715x over previous
import functools

import jax
import jax.numpy as jnp
from jax import lax
from jax.experimental import pallas as pl
from jax.experimental.pallas import tpu as pltpu
from jax.experimental.pallas import tpu_sc as plsc

N = 50000
E = 800000
PERIODS = 12
HID = 32

NP = 50176
NPT = NP // 16
EP = 819200
ROWS = EP // 128
NW = 32
ROWS_W = ROWS // NW
KB = 40
NG = ROWS_W // KB

_mesh = plsc.VectorSubcoreMesh(core_axis_name="c", subcore_axis_name="s")


@functools.partial(
    pl.kernel,
    out_type=jax.ShapeDtypeStruct((2 * NP,), jnp.float32),
    mesh=_mesh,
    scratch_types=[
        pltpu.VMEM_SHARED((NP,), jnp.float32),
        pltpu.VMEM((KB, 128), jnp.int32),
        pltpu.VMEM((KB, 128), jnp.float32),
        pltpu.VMEM((NPT,), jnp.float32),
    ],
    compiler_params=pltpu.CompilerParams(needs_layout_passes=False),
)
def _deg_kernel(dst_hbm, ew_hbm, out_hbm, deg_sh, dstbuf, ewbuf, zbuf):
    c = lax.axis_index("c")
    s = lax.axis_index("s")
    wid = s * 2 + c
    zero16 = jnp.zeros((16,), jnp.float32)

    def zb(i, _):
        zbuf[pl.ds(i * 16, 16)] = zero16
        return 0

    lax.fori_loop(0, NPT // 16, zb, 0)
    pltpu.sync_copy(zbuf, deg_sh.at[pl.ds(s * NPT, NPT)])
    plsc.subcore_barrier()

    def grp(g, _):
        base = wid * ROWS_W + g * KB
        pltpu.sync_copy(dst_hbm.at[pl.ds(base, KB)], dstbuf)
        pltpu.sync_copy(ew_hbm.at[pl.ds(base, KB)], ewbuf)

        def row(r, _):
            pltpu.sync_copy(ewbuf.at[r], deg_sh.at[dstbuf.at[r]], add=True)
            return 0

        lax.fori_loop(0, KB, row, 0)
        return 0

    lax.fori_loop(0, NG, grp, 0)
    plsc.subcore_barrier()
    off = pl.multiple_of(c * NP + s * NPT, 8)
    pltpu.sync_copy(deg_sh.at[pl.ds(s * NPT, NPT)], zbuf)
    pltpu.sync_copy(zbuf, out_hbm.at[pl.ds(off, NPT)])


def _dinv_body(deg_ref, out_ref):
    out_ref[...] = lax.rsqrt(deg_ref[0] + deg_ref[1] + 1.0)


_dinv_call = pl.pallas_call(
    _dinv_body,
    out_shape=jax.ShapeDtypeStruct((NP // 128, 128), jnp.float32),
)


@functools.partial(
    pl.kernel,
    out_type=jax.ShapeDtypeStruct((2, NP, 16), jnp.float32),
    mesh=_mesh,
    scratch_types=[
        pltpu.VMEM_SHARED((NP, 16), jnp.float32),
        pltpu.VMEM_SHARED((NP,), jnp.float32),
        pltpu.VMEM((KB, 128), jnp.int32),
        pltpu.VMEM((KB, 128), jnp.int32),
        pltpu.VMEM((KB, 128), jnp.float32),
        pltpu.VMEM((4, 128), jnp.float32),
        pltpu.VMEM((4, 128, 16), jnp.float32),
        pltpu.VMEM((2, 128, 16), jnp.float32),
        pltpu.VMEM((112, 16), jnp.float32),
        pltpu.VMEM((NPT // 2,), jnp.float32),
        pltpu.SemaphoreType.DMA((4,)),
        pltpu.SemaphoreType.DMA((4,)),
        pltpu.SemaphoreType.DMA((2,)),
    ],
    compiler_params=pltpu.CompilerParams(
        needs_layout_passes=False, use_tc_tiling_on_sc=False),
)
def _edge_kernel(src_hbm, dst_hbm, ew_hbm, dinv_hbm, x_hbm, out_hbm,
                 s_sh, dinv_sh, srcbuf, dstbuf, ewbuf, dg, xg, xs, zbuf, dbuf,
                 dsem, xsem, ssem):
    c = lax.axis_index("c")
    s = lax.axis_index("s")
    wid = s * 2 + c
    zero16 = jnp.zeros((16,), jnp.float32)

    def zb(i, _):
        zbuf[i, :] = zero16
        return 0

    lax.fori_loop(0, 112, zb, 0)
    for q in range(28):
        pltpu.sync_copy(zbuf, s_sh.at[pl.ds(s * NPT + q * 112, 112), :])
    for q in range(2):
        pltpu.sync_copy(dinv_hbm.at[pl.ds(s * NPT + q * (NPT // 2), NPT // 2)], dbuf)
        pltpu.sync_copy(dbuf, dinv_sh.at[pl.ds(s * NPT + q * (NPT // 2), NPT // 2)])
    plsc.subcore_barrier()

    def _issue_gathers(r, b):
        pltpu.async_copy(dinv_sh.at[srcbuf.at[r]], dg.at[b], dsem.at[b])
        pltpu.async_copy(x_hbm.at[srcbuf.at[r]], xg.at[b], xsem.at[b])

    def grp(g, _):
        base = wid * ROWS_W + g * KB
        pltpu.sync_copy(src_hbm.at[pl.ds(base, KB)], srcbuf)
        pltpu.sync_copy(dst_hbm.at[pl.ds(base, KB)], dstbuf)
        pltpu.sync_copy(ew_hbm.at[pl.ds(base, KB)], ewbuf)
        for r0 in range(4):
            _issue_gathers(r0, r0)

        def row(r, _):
            b = r % 4
            sb = r % 2
            pltpu.make_async_copy(dinv_sh.at[srcbuf.at[r]], dg.at[b],
                                  dsem.at[b]).wait()
            pltpu.make_async_copy(x_hbm.at[srcbuf.at[r]], xg.at[b],
                                  xsem.at[b]).wait()

            @pl.when(r >= 2)
            def _():
                pltpu.make_async_copy(xs.at[sb], s_sh.at[dstbuf.at[r]],
                                      ssem.at[sb]).wait()

            def sc(j, _):
                jb = j * 16
                w16 = dg[b, pl.ds(jb, 16)] * ewbuf[r, pl.ds(jb, 16)]
                for u in range(16):
                    xs[sb, jb + u, :] = xg[b, jb + u, :] * w16[u]
                return 0

            lax.fori_loop(0, 8, sc, 0)
            pltpu.async_copy(xs.at[sb], s_sh.at[dstbuf.at[r]], ssem.at[sb],
                             add=True)

            @pl.when(r + 4 < KB)
            def _():
                _issue_gathers(r + 4, b)

            return 0

        lax.fori_loop(0, KB, row, 0)
        pltpu.make_async_copy(xs.at[0], s_sh.at[dstbuf.at[0]], ssem.at[0]).wait()
        pltpu.make_async_copy(xs.at[1], s_sh.at[dstbuf.at[1]], ssem.at[1]).wait()
        return 0

    lax.fori_loop(0, NG, grp, 0)
    plsc.subcore_barrier()
    for q in range(28):
        pltpu.sync_copy(s_sh.at[pl.ds(s * NPT + q * 112, 112), :], zbuf)
        pltpu.sync_copy(zbuf, out_hbm.at[c, pl.ds(s * NPT + q * 112, 112), :])


BB = 1792
NBLK = NP // BB


def _dense_body(p_ref, st_ref, xt_ref, dinv_ref, out_ref):
    dinv = dinv_ref[...]
    t = st_ref[0] + st_ref[1]
    sfull = dinv * t + (dinv * dinv) * xt_ref[...]
    az = p_ref[0, :].reshape(HID, 1)
    cz = p_ref[1, :].reshape(HID, 1)
    ah = p_ref[2, :].reshape(HID, 1)
    ch = p_ref[3, :].reshape(HID, 1)
    wl = p_ref[4, :].reshape(HID, 1)
    acc = jnp.zeros((HID, BB), jnp.float32)
    for p in range(PERIODS):
        sp = sfull[p:p + 1, :]
        z = jax.nn.sigmoid(az * sp + cz)
        ht = jnp.tanh(ah * sp + ch)
        acc = acc + p_ref[5, p] * ((1.0 - z) * ht)
    h = jnp.maximum(acc, 0.0)
    out_ref[...] = jnp.sum(h * wl, axis=0, keepdims=True) + p_ref[6, 0]


_dense_call = pl.pallas_call(
    _dense_body,
    grid=(NBLK,),
    in_specs=[
        pl.BlockSpec((8, HID), lambda i: (0, 0)),
        pl.BlockSpec((2, 16, BB), lambda i: (0, 0, i)),
        pl.BlockSpec((16, BB), lambda i: (0, i)),
        pl.BlockSpec((1, BB), lambda i: (0, i)),
    ],
    out_specs=pl.BlockSpec((1, BB), lambda i: (0, i)),
    out_shape=jax.ShapeDtypeStruct((1, NP), jnp.float32),
)


def kernel(x, edge_index, edge_weight, Wcz, bcz, Wcr, bcr, Wch, bch,
           Wlz, blz, Wlr, blr, Wlh, blh, att, Wlin, blin):
    src = edge_index[0]
    dst = edge_index[1]
    pad = EP - E
    src_p = jnp.concatenate([src, jnp.zeros((pad,), jnp.int32)])
    dst_p = jnp.concatenate([dst, (jnp.arange(pad, dtype=jnp.int32) * 41) % N])
    ew_p = jnp.concatenate([edge_weight, jnp.zeros((pad,), jnp.float32)])
    src2 = src_p.reshape(ROWS, 128)
    dst2 = dst_p.reshape(ROWS, 128)
    ew2 = ew_p.reshape(ROWS, 128)
    x_pad = jnp.pad(x, ((0, NP - N), (0, 16 - PERIODS)))

    deg2 = _deg_kernel(dst2, ew2)
    dinv = _dinv_call(deg2.reshape(2, NP // 128, 128)).reshape(NP)
    s2 = _edge_kernel(src2, dst2, ew2, dinv, x_pad)

    wlz_t = Wlz[:HID]
    wlh_t = Wlh[:HID]
    az = (Wcz @ wlz_t)[0]
    cz = bcz @ wlz_t + blz
    ah = (Wch @ wlh_t)[0]
    ch = bch @ wlh_t + blh
    probs = jax.nn.softmax(att)
    params = jnp.stack([
        az, cz, ah, ch, Wlin[:, 0],
        jnp.pad(probs, (0, HID - PERIODS)),
        jnp.full((HID,), blin[0], jnp.float32),
        jnp.zeros((HID,), jnp.float32),
    ]).astype(jnp.float32)

    st2 = jnp.transpose(s2, (0, 2, 1))
    out_t = _dense_call(params, st2, x_pad.T, dinv.reshape(1, NP))
    return out_t[0, :N].reshape(N, 1)
